# baseline (speedup 1.0000x reference)
import functools

import jax
import jax.numpy as jnp
from jax import lax
from jax.experimental import pallas as pl
from jax.experimental.pallas import tpu as pltpu
from jax.experimental.pallas import tpu_sc as plsc

N = 10000
E = 320000
D = 128
L = 2

NC = 2
NS = 16
NW = NC * NS
CHUNK = 64
CPW = 160
E2 = NW * CPW * CHUNK
NB = 2
NQ = 4


def _lane_bcast(v, t):
    return lax.gather(
        v, jnp.full((16, 1), t, jnp.int32),
        dimension_numbers=lax.GatherDimensionNumbers(
            offset_dims=(), collapsed_slice_dims=(0,), start_index_map=(0,)),
        slice_sizes=(1,),
        mode=lax.GatherScatterMode.PROMISE_IN_BOUNDS)


def _sc_degree(col2d, dat2d):
    mesh = plsc.VectorSubcoreMesh(core_axis_name="c", subcore_axis_name="s")

    @functools.partial(
        pl.kernel,
        out_type=jax.ShapeDtypeStruct((NC * N,), jnp.float32),
        mesh=mesh,
        scratch_types=[
            pltpu.VMEM_SHARED((N,), jnp.float32),
            pltpu.VMEM((CPW, CHUNK), jnp.int32),
            pltpu.VMEM((CPW, CHUNK), jnp.float32),
            pltpu.VMEM((640,), jnp.float32),
            pltpu.SemaphoreType.DMA,
            pltpu.SemaphoreType.DMA,
        ],
    )
    def k(col_hbm, dat_hbm, out_hbm, acc, colb, datb, zbuf, sem_p, sem_s):
        c = lax.axis_index("c")
        s = lax.axis_index("s")
        wid = s * NC + c
        blk = wid * CPW

        p1 = pltpu.async_copy(col_hbm.at[pl.ds(blk, CPW)], colb, sem_p)
        p2 = pltpu.async_copy(dat_hbm.at[pl.ds(blk, CPW)], datb, sem_p)

        z = jnp.zeros((16,), jnp.float32)

        @pl.loop(0, 640, step=16)
        def _(i):
            zbuf[pl.ds(i, 16)] = z

        pltpu.sync_copy(zbuf.at[pl.ds(0, 624)], acc.at[pl.ds(s * 624, 624)])

        @pl.when(s == NS - 1)
        def _():
            pltpu.sync_copy(zbuf.at[pl.ds(0, 16)], acc.at[pl.ds(9984, 16)])

        p1.wait()
        p2.wait()
        plsc.subcore_barrier()

        @pl.loop(0, CPW // 8)
        def _(g):
            descs = []
            for u in range(8):
                i = g * 8 + u
                descs.append(pltpu.async_copy(
                    datb.at[i], acc.at[colb.at[i]], sem_s, add=True))
            for d in descs:
                d.wait()

        plsc.subcore_barrier()
        pltpu.sync_copy(acc.at[pl.ds(s * 624, 624)], zbuf.at[pl.ds(0, 624)])
        pltpu.sync_copy(zbuf.at[pl.ds(0, 624)],
                        out_hbm.at[pl.ds(c * N + s * 624, 624)])

        @pl.when(s == NS - 1)
        def _():
            pltpu.sync_copy(acc.at[pl.ds(9984, 16)], zbuf.at[pl.ds(0, 16)])
            pltpu.sync_copy(zbuf.at[pl.ds(0, 16)],
                            out_hbm.at[pl.ds(c * N + 9984, 16)])

    return k(col2d, dat2d)


def _sc_spmm(row1d, col1d, dat1d, deg, h):
    mesh = plsc.VectorSubcoreMesh(core_axis_name="c", subcore_axis_name="s")

    idx_slots = [
        pltpu.VMEM((CHUNK,), jnp.int32) for _ in range(2 * NQ)
    ] + [pltpu.VMEM((CHUNK,), jnp.float32) for _ in range(NQ)]

    @functools.partial(
        pl.kernel,
        out_type=jax.ShapeDtypeStruct((NC, N, D), jnp.float32),
        mesh=mesh,
        scratch_types=[
            pltpu.VMEM_SHARED((N + 64, D), jnp.float32),
        ] + idx_slots + [
            pltpu.VMEM((CHUNK, D), jnp.float32),
            pltpu.VMEM((CHUNK, D), jnp.float32),
            pltpu.VMEM((CHUNK, D), jnp.float32),
            pltpu.VMEM((CHUNK, D), jnp.float32),
            pltpu.VMEM((CHUNK,), jnp.float32),
            pltpu.VMEM((CHUNK,), jnp.float32),
            pltpu.VMEM((CHUNK,), jnp.float32),
            pltpu.VMEM((CHUNK,), jnp.float32),
            pltpu.SemaphoreType.DMA,
            pltpu.SemaphoreType.DMA,
            pltpu.SemaphoreType.DMA,
            pltpu.SemaphoreType.DMA,
            pltpu.SemaphoreType.DMA,
            pltpu.SemaphoreType.DMA,
            pltpu.SemaphoreType.DMA,
            pltpu.SemaphoreType.DMA,
        ],
    )
    def k(row_hbm, col_hbm, dat_hbm, deg_hbm, h_hbm, out_hbm, acc,
          rv0, rv1, rv2, rv3, cv0, cv1, cv2, cv3, dv0, dv1, dv2, dv3,
          rg0, rg1, rs0, rs1, dg0, dg1, sc0, sc1,
          sem_g0, sem_g1, sem_s0, sem_s1, si0, si1, si2, si3):
        rowv = (rv0, rv1, rv2, rv3)
        colv = (cv0, cv1, cv2, cv3)
        datv = (dv0, dv1, dv2, dv3)
        rg = (rg0, rg1)
        rs = (rs0, rs1)
        dg = (dg0, dg1)
        sc = (sc0, sc1)
        sem_g = (sem_g0, sem_g1)
        sem_s = (sem_s0, sem_s1)
        sem_i = (si0, si1, si2, si3)

        c = lax.axis_index("c")
        s = lax.axis_index("s")
        wid = s * NC + c
        base = wid * CPW * CHUNK

        def issue_idx(cc, q):
            off = base + cc * CHUNK
            pltpu.async_copy(row_hbm.at[pl.ds(off, CHUNK)], rowv[q], sem_i[q])
            pltpu.async_copy(col_hbm.at[pl.ds(off, CHUNK)], colv[q], sem_i[q])
            pltpu.async_copy(dat_hbm.at[pl.ds(off, CHUNK)], datv[q], sem_i[q])

        def wait_idx(cc, q):
            off = base + cc * CHUNK
            pltpu.make_async_copy(row_hbm.at[pl.ds(off, CHUNK)], rowv[q],
                                  sem_i[q]).wait()
            pltpu.make_async_copy(col_hbm.at[pl.ds(off, CHUNK)], colv[q],
                                  sem_i[q]).wait()
            pltpu.make_async_copy(dat_hbm.at[pl.ds(off, CHUNK)], datv[q],
                                  sem_i[q]).wait()

        def issue_gather(b, q):
            pltpu.async_copy(h_hbm.at[colv[q]], rg[b], sem_g[b])
            pltpu.async_copy(deg_hbm.at[colv[q]], dg[b], sem_g[b])

        def wait_gather(b, q):
            pltpu.make_async_copy(h_hbm.at[colv[q]], rg[b], sem_g[b]).wait()
            pltpu.make_async_copy(deg_hbm.at[colv[q]], dg[b], sem_g[b]).wait()

        def issue_scatter(b, q):
            pltpu.async_copy(rs[b], acc.at[rowv[q]], sem_s[b], add=True)

        def wait_scatter(b, q):
            pltpu.make_async_copy(rs[b], acc.at[rowv[q]], sem_s[b]).wait()

        def scale_chunk(b, q):
            @pl.loop(0, CHUNK // 16)
            def _(j):
                sc[b][pl.ds(j * 16, 16)] = (
                    datv[q][pl.ds(j * 16, 16)] / dg[b][pl.ds(j * 16, 16)])

            @pl.loop(0, CHUNK // 16)
            def _(jj):
                sj = sc[b][pl.ds(jj * 16, 16)]
                for t in range(16):
                    e = jj * 16 + t
                    vv = _lane_bcast(sj, t)
                    for d in range(D // 16):
                        rs[b][e, pl.ds(d * 16, 16)] = (
                            rg[b][e, pl.ds(d * 16, 16)] * vv)

        def chunk_step(cc, b, q, first, more):
            wait_gather(b, q)
            if not first:
                wait_scatter(b, (q + 2) % NQ)
                if more:
                    issue_idx(cc + 2, (q + 2) % NQ)
            scale_chunk(b, q)
            if more:
                wait_idx(cc + 2, (q + 2) % NQ)
                issue_gather(b, (q + 2) % NQ)
            issue_scatter(b, q)

        for q in range(NQ):
            issue_idx(q, q)
        for b in range(NB):
            wait_idx(b, b)
            issue_gather(b, b)

        z = jnp.zeros((16,), jnp.float32)

        @pl.loop(0, CHUNK)
        def _(i):
            for d in range(D // 16):
                rs0[i, pl.ds(d * 16, 16)] = z

        @pl.loop(0, 9)
        def _(i):
            pltpu.sync_copy(rs0, acc.at[pl.ds(s * 624 + i * 64, 64)])

        pltpu.sync_copy(rs0.at[pl.ds(0, 48)], acc.at[pl.ds(s * 624 + 576, 48)])

        @pl.when(s == NS - 1)
        def _():
            pltpu.sync_copy(rs0.at[pl.ds(0, 16)], acc.at[pl.ds(9984, 16)])

        plsc.subcore_barrier()

        chunk_step(0, 0, 0, first=True, more=True)
        chunk_step(1, 1, 1, first=True, more=True)

        @pl.loop(0, (CPW - 4) // NQ)
        def _(g):
            cc = 2 + g * NQ
            for kk in range(NQ):
                chunk_step(cc + kk, (2 + kk) % NB, (2 + kk) % NQ,
                           first=False, more=True)

        chunk_step(CPW - 2, 0, 2, first=False, more=False)
        chunk_step(CPW - 1, 1, 3, first=False, more=False)
        wait_scatter(0, 2)
        wait_scatter(1, 3)

        plsc.subcore_barrier()

        @pl.loop(0, 9)
        def _(i):
            r0 = s * 624 + i * 64
            pltpu.sync_copy(acc.at[pl.ds(r0, 64)], rs0)
            pltpu.sync_copy(rs0, out_hbm.at[c, pl.ds(r0, 64)])

        pltpu.sync_copy(acc.at[pl.ds(s * 624 + 576, 48)], rs0.at[pl.ds(0, 48)])
        pltpu.sync_copy(rs0.at[pl.ds(0, 48)],
                        out_hbm.at[c, pl.ds(s * 624 + 576, 48)])

        @pl.when(s == NS - 1)
        def _():
            pltpu.sync_copy(acc.at[pl.ds(9984, 16)], rs0.at[pl.ds(0, 16)])
            pltpu.sync_copy(rs0.at[pl.ds(0, 16)],
                            out_hbm.at[c, pl.ds(9984, 16)])

    return k(row1d, col1d, dat1d, deg, h)


def _tc1_body(dp_ref, emb_ref, w1t_ref, wa_ref, ba_ref, deg_ref, h_ref):
    deg_ref[...] = dp_ref[0] + dp_ref[1]
    x = emb_ref[...]
    scores = jnp.dot(x, wa_ref[...], preferred_element_type=jnp.float32)
    scores = scores + ba_ref[0, 0]
    m = jnp.max(scores)
    ex = jnp.exp(scores - m)
    attn = ex / jnp.sum(ex)
    xw = jnp.dot(x, w1t_ref[...], preferred_element_type=jnp.float32)
    h_ref[...] = xw * attn


def _tc1(deg_partials, emb, w1t, wa, ba):
    return pl.pallas_call(
        _tc1_body,
        out_shape=(jax.ShapeDtypeStruct((N,), jnp.float32),
                   jax.ShapeDtypeStruct((N, D), jnp.float32)),
    )(deg_partials, emb, w1t, wa, ba)


def _norm_rows(x):
    nrm = jnp.sqrt(jnp.sum(x * x, axis=-1, keepdims=True))
    return x / jnp.maximum(nrm, 1e-12)


def _tc2_body(p_ref, emb_ref, w2t_ref, wa_ref, ba_ref, f1_ref, h_ref):
    x = _norm_rows(p_ref[0] + p_ref[1])
    f1_ref[...] = emb_ref[...] + x
    scores = jnp.dot(x, wa_ref[...], preferred_element_type=jnp.float32)
    scores = scores + ba_ref[0, 0]
    m = jnp.max(scores)
    ex = jnp.exp(scores - m)
    attn = ex / jnp.sum(ex)
    xw = jnp.dot(x, w2t_ref[...], preferred_element_type=jnp.float32)
    h_ref[...] = xw * attn


def _tc2(partials, emb, w2t, wa, ba):
    return pl.pallas_call(
        _tc2_body,
        out_shape=(jax.ShapeDtypeStruct((N, D), jnp.float32),
                   jax.ShapeDtypeStruct((N, D), jnp.float32)),
    )(partials, emb, w2t, wa, ba)


def _tc3_body(p_ref, f1_ref, out_ref):
    x = _norm_rows(p_ref[0] + p_ref[1])
    out_ref[...] = (f1_ref[...] + x) * (1.0 / (L + 1))


def _tc3(partials, f1):
    return pl.pallas_call(
        _tc3_body,
        out_shape=jax.ShapeDtypeStruct((N, D), jnp.float32),
    )(partials, f1)


def kernel(adj_row, adj_col, adj_data, embedding, W_item, W_att, b_att):
    pad = E2 - E
    trash = N + (jnp.arange(pad, dtype=jnp.int32) % 64)
    row1d = jnp.concatenate([adj_row.astype(jnp.int32), trash])
    col1d = jnp.concatenate(
        [adj_col.astype(jnp.int32), jnp.zeros((pad,), jnp.int32)])
    dat1d = jnp.concatenate([adj_data, jnp.zeros((pad,), jnp.float32)])
    col2d = col1d.reshape(E2 // CHUNK, CHUNK)
    dat2d = dat1d.reshape(E2 // CHUNK, CHUNK)
    ba = b_att.reshape(1, 1)

    deg_partials = _sc_degree(col2d, dat2d).reshape(NC, N)
    deg, h1 = _tc1(deg_partials, embedding, W_item[0].T, W_att, ba)
    p1 = _sc_spmm(row1d, col1d, dat1d, deg, h1)
    f1, h2 = _tc2(p1, embedding, W_item[1].T, W_att, ba)
    p2 = _sc_spmm(row1d, col1d, dat1d, deg, h2)
    return _tc3(p2, f1)


# speedup vs baseline: 2.1066x; 2.1066x over previous
import functools

import jax
import jax.numpy as jnp
from jax import lax
from jax.experimental import pallas as pl
from jax.experimental.pallas import tpu as pltpu
from jax.experimental.pallas import tpu_sc as plsc

N = 10000
E = 320000
D = 128
L = 2

NC = 2
NS = 16
NW = NC * NS
CHUNK = 64
CPW = 160
E2 = NW * CPW * CHUNK
NB = 2
NQ = 4


def _lane_bcast(v, t):
    return lax.gather(
        v, jnp.full((16, 1), t, jnp.int32),
        dimension_numbers=lax.GatherDimensionNumbers(
            offset_dims=(), collapsed_slice_dims=(0,), start_index_map=(0,)),
        slice_sizes=(1,),
        mode=lax.GatherScatterMode.PROMISE_IN_BOUNDS)


def _sc_degree(col2d, dat2d):
    mesh = plsc.VectorSubcoreMesh(core_axis_name="c", subcore_axis_name="s")

    @functools.partial(
        pl.kernel,
        out_type=jax.ShapeDtypeStruct((NC * N,), jnp.float32),
        mesh=mesh,
        scratch_types=[
            pltpu.VMEM_SHARED((N,), jnp.float32),
            pltpu.VMEM((CPW, CHUNK), jnp.int32),
            pltpu.VMEM((CPW, CHUNK), jnp.float32),
            pltpu.VMEM((640,), jnp.float32),
            pltpu.SemaphoreType.DMA,
            pltpu.SemaphoreType.DMA,
        ],
    )
    def k(col_hbm, dat_hbm, out_hbm, acc, colb, datb, zbuf, sem_p, sem_s):
        c = lax.axis_index("c")
        s = lax.axis_index("s")
        wid = s * NC + c
        blk = wid * CPW

        p1 = pltpu.async_copy(col_hbm.at[pl.ds(blk, CPW)], colb, sem_p)
        p2 = pltpu.async_copy(dat_hbm.at[pl.ds(blk, CPW)], datb, sem_p)

        z = jnp.zeros((16,), jnp.float32)

        @pl.loop(0, 640, step=16)
        def _(i):
            zbuf[pl.ds(i, 16)] = z

        pltpu.sync_copy(zbuf.at[pl.ds(0, 624)], acc.at[pl.ds(s * 624, 624)])

        @pl.when(s == NS - 1)
        def _():
            pltpu.sync_copy(zbuf.at[pl.ds(0, 16)], acc.at[pl.ds(9984, 16)])

        p1.wait()
        p2.wait()
        plsc.subcore_barrier()

        @pl.loop(0, CPW // 8)
        def _(g):
            descs = []
            for u in range(8):
                i = g * 8 + u
                descs.append(pltpu.async_copy(
                    datb.at[i], acc.at[colb.at[i]], sem_s, add=True))
            for d in descs:
                d.wait()

        plsc.subcore_barrier()
        pltpu.sync_copy(acc.at[pl.ds(s * 624, 624)], zbuf.at[pl.ds(0, 624)])
        pltpu.sync_copy(zbuf.at[pl.ds(0, 624)],
                        out_hbm.at[pl.ds(c * N + s * 624, 624)])

        @pl.when(s == NS - 1)
        def _():
            pltpu.sync_copy(acc.at[pl.ds(9984, 16)], zbuf.at[pl.ds(0, 16)])
            pltpu.sync_copy(zbuf.at[pl.ds(0, 16)],
                            out_hbm.at[pl.ds(c * N + 9984, 16)])

    return k(col2d, dat2d)


def _sc_spmm(row1d, col1d, dat1d, deg, h):
    mesh = plsc.VectorSubcoreMesh(core_axis_name="c", subcore_axis_name="s")

    idx_slots = [
        pltpu.VMEM((CHUNK,), jnp.int32) for _ in range(2 * NQ)
    ] + [pltpu.VMEM((CHUNK,), jnp.float32) for _ in range(NQ)]

    @functools.partial(
        pl.kernel,
        out_type=jax.ShapeDtypeStruct((NC, N, D), jnp.float32),
        mesh=mesh,
        scratch_types=[
            pltpu.VMEM_SHARED((N + 64, D), jnp.float32),
        ] + idx_slots + [
            pltpu.VMEM((CHUNK, D), jnp.float32),
            pltpu.VMEM((CHUNK, D), jnp.float32),
            pltpu.VMEM((CHUNK, D), jnp.float32),
            pltpu.VMEM((CHUNK, D), jnp.float32),
            pltpu.VMEM((CHUNK,), jnp.float32),
            pltpu.VMEM((CHUNK,), jnp.float32),
            pltpu.VMEM((CHUNK,), jnp.float32),
            pltpu.VMEM((CHUNK,), jnp.float32),
            pltpu.SemaphoreType.DMA,
            pltpu.SemaphoreType.DMA,
            pltpu.SemaphoreType.DMA,
            pltpu.SemaphoreType.DMA,
            pltpu.SemaphoreType.DMA,
            pltpu.SemaphoreType.DMA,
            pltpu.SemaphoreType.DMA,
            pltpu.SemaphoreType.DMA,
        ],
    )
    def k(row_hbm, col_hbm, dat_hbm, deg_hbm, h_hbm, out_hbm, acc,
          rv0, rv1, rv2, rv3, cv0, cv1, cv2, cv3, dv0, dv1, dv2, dv3,
          rg0, rg1, rs0, rs1, dg0, dg1, sc0, sc1,
          sem_g0, sem_g1, sem_s0, sem_s1, si0, si1, si2, si3):
        rowv = (rv0, rv1, rv2, rv3)
        colv = (cv0, cv1, cv2, cv3)
        datv = (dv0, dv1, dv2, dv3)
        rg = (rg0, rg1)
        rs = (rs0, rs1)
        dg = (dg0, dg1)
        sc = (sc0, sc1)
        sem_g = (sem_g0, sem_g1)
        sem_s = (sem_s0, sem_s1)
        sem_i = (si0, si1, si2, si3)

        c = lax.axis_index("c")
        s = lax.axis_index("s")
        wid = s * NC + c
        base = wid * CPW * CHUNK

        def issue_idx(cc, q):
            off = base + cc * CHUNK
            pltpu.async_copy(row_hbm.at[pl.ds(off, CHUNK)], rowv[q], sem_i[q])
            pltpu.async_copy(col_hbm.at[pl.ds(off, CHUNK)], colv[q], sem_i[q])
            pltpu.async_copy(dat_hbm.at[pl.ds(off, CHUNK)], datv[q], sem_i[q])

        def wait_idx(cc, q):
            off = base + cc * CHUNK
            pltpu.make_async_copy(row_hbm.at[pl.ds(off, CHUNK)], rowv[q],
                                  sem_i[q]).wait()
            pltpu.make_async_copy(col_hbm.at[pl.ds(off, CHUNK)], colv[q],
                                  sem_i[q]).wait()
            pltpu.make_async_copy(dat_hbm.at[pl.ds(off, CHUNK)], datv[q],
                                  sem_i[q]).wait()

        def issue_gather(b, q):
            pltpu.async_copy(h_hbm.at[colv[q]], rg[b], sem_g[b])
            pltpu.async_copy(deg_hbm.at[colv[q]], dg[b], sem_g[b])

        def wait_gather(b, q):
            pltpu.make_async_copy(h_hbm.at[colv[q]], rg[b], sem_g[b]).wait()
            pltpu.make_async_copy(deg_hbm.at[colv[q]], dg[b], sem_g[b]).wait()

        def issue_scatter(b, q):
            pltpu.async_copy(rs[b], acc.at[rowv[q]], sem_s[b], add=True)

        def wait_scatter(b, q):
            pltpu.make_async_copy(rs[b], acc.at[rowv[q]], sem_s[b]).wait()

        def scale_chunk(b, q):
            @pl.loop(0, CHUNK // 16)
            def _(j):
                sc[b][pl.ds(j * 16, 16)] = (
                    datv[q][pl.ds(j * 16, 16)] / dg[b][pl.ds(j * 16, 16)])

            @pl.loop(0, CHUNK // 16)
            def _(jj):
                sj = sc[b][pl.ds(jj * 16, 16)]
                for t in range(16):
                    e = jj * 16 + t
                    vv = _lane_bcast(sj, t)
                    for d in range(D // 16):
                        rs[b][e, pl.ds(d * 16, 16)] = (
                            rg[b][e, pl.ds(d * 16, 16)] * vv)

        def chunk_step(cc, b, q, first, more):
            wait_gather(b, q)
            if not first:
                wait_scatter(b, (q + 2) % NQ)
                if more:
                    issue_idx(cc + 2, (q + 2) % NQ)
            scale_chunk(b, q)
            if more:
                wait_idx(cc + 2, (q + 2) % NQ)
                issue_gather(b, (q + 2) % NQ)
            issue_scatter(b, q)

        for q in range(NQ):
            issue_idx(q, q)
        for b in range(NB):
            wait_idx(b, b)
            issue_gather(b, b)

        z = jnp.zeros((16,), jnp.float32)

        @pl.loop(0, CHUNK)
        def _(i):
            for d in range(D // 16):
                rs0[i, pl.ds(d * 16, 16)] = z

        @pl.loop(0, 9)
        def _(i):
            pltpu.sync_copy(rs0, acc.at[pl.ds(s * 624 + i * 64, 64)])

        pltpu.sync_copy(rs0.at[pl.ds(0, 48)], acc.at[pl.ds(s * 624 + 576, 48)])

        @pl.when(s == NS - 1)
        def _():
            pltpu.sync_copy(rs0.at[pl.ds(0, 16)], acc.at[pl.ds(9984, 16)])

        plsc.subcore_barrier()

        chunk_step(0, 0, 0, first=True, more=True)
        chunk_step(1, 1, 1, first=True, more=True)

        @pl.loop(0, (CPW - 4) // NQ)
        def _(g):
            cc = 2 + g * NQ
            for kk in range(NQ):
                chunk_step(cc + kk, (2 + kk) % NB, (2 + kk) % NQ,
                           first=False, more=True)

        chunk_step(CPW - 2, 0, 2, first=False, more=False)
        chunk_step(CPW - 1, 1, 3, first=False, more=False)
        wait_scatter(0, 2)
        wait_scatter(1, 3)

        plsc.subcore_barrier()

        @pl.loop(0, 9)
        def _(i):
            r0 = s * 624 + i * 64
            pltpu.sync_copy(acc.at[pl.ds(r0, 64)], rs0)
            pltpu.sync_copy(rs0, out_hbm.at[c, pl.ds(r0, 64)])

        pltpu.sync_copy(acc.at[pl.ds(s * 624 + 576, 48)], rs0.at[pl.ds(0, 48)])
        pltpu.sync_copy(rs0.at[pl.ds(0, 48)],
                        out_hbm.at[c, pl.ds(s * 624 + 576, 48)])

        @pl.when(s == NS - 1)
        def _():
            pltpu.sync_copy(acc.at[pl.ds(9984, 16)], rs0.at[pl.ds(0, 16)])
            pltpu.sync_copy(rs0.at[pl.ds(0, 16)],
                            out_hbm.at[c, pl.ds(9984, 16)])

    return k(row1d, col1d, dat1d, deg, h)


def _tc1_body(dp_ref, emb_ref, w1t_ref, wa_ref, ba_ref, deg_ref, h_ref):
    deg_ref[...] = dp_ref[0] + dp_ref[1]
    x = emb_ref[...]
    scores = jnp.dot(x, wa_ref[...], preferred_element_type=jnp.float32)
    scores = scores + ba_ref[0, 0]
    m = jnp.max(scores)
    ex = jnp.exp(scores - m)
    attn = ex / jnp.sum(ex)
    xw = jnp.dot(x, w1t_ref[...], preferred_element_type=jnp.float32)
    h_ref[...] = xw * attn


def _tc1(deg_partials, emb, w1t, wa, ba):
    return pl.pallas_call(
        _tc1_body,
        out_shape=(jax.ShapeDtypeStruct((N,), jnp.float32),
                   jax.ShapeDtypeStruct((N, D), jnp.float32)),
    )(deg_partials, emb, w1t, wa, ba)


def _norm_rows(x):
    nrm = jnp.sqrt(jnp.sum(x * x, axis=-1, keepdims=True))
    return x / jnp.maximum(nrm, 1e-12)


def _tc2_body(p_ref, emb_ref, w2t_ref, wa_ref, ba_ref, f1_ref, h_ref):
    x = _norm_rows(p_ref[0] + p_ref[1])
    f1_ref[...] = emb_ref[...] + x
    scores = jnp.dot(x, wa_ref[...], preferred_element_type=jnp.float32)
    scores = scores + ba_ref[0, 0]
    m = jnp.max(scores)
    ex = jnp.exp(scores - m)
    attn = ex / jnp.sum(ex)
    xw = jnp.dot(x, w2t_ref[...], preferred_element_type=jnp.float32)
    h_ref[...] = xw * attn


def _tc2(partials, emb, w2t, wa, ba):
    return pl.pallas_call(
        _tc2_body,
        out_shape=(jax.ShapeDtypeStruct((N, D), jnp.float32),
                   jax.ShapeDtypeStruct((N, D), jnp.float32)),
    )(partials, emb, w2t, wa, ba)


def _tc3_body(p_ref, f1_ref, out_ref):
    x = _norm_rows(p_ref[0] + p_ref[1])
    out_ref[...] = (f1_ref[...] + x) * (1.0 / (L + 1))


def _tc3(partials, f1):
    return pl.pallas_call(
        _tc3_body,
        out_shape=jax.ShapeDtypeStruct((N, D), jnp.float32),
    )(partials, f1)


def kernel(adj_row, adj_col, adj_data, embedding, W_item, W_att, b_att):
    pad = E2 - E
    trash = N + (jnp.arange(pad, dtype=jnp.int32) % 64)
    row1d = jnp.concatenate([adj_row.astype(jnp.int32), trash])
    col1d = jnp.concatenate(
        [adj_col.astype(jnp.int32), jnp.arange(pad, dtype=jnp.int32) % N])
    dat1d = jnp.concatenate([adj_data, jnp.zeros((pad,), jnp.float32)])
    col2d = col1d.reshape(E2 // CHUNK, CHUNK)
    dat2d = dat1d.reshape(E2 // CHUNK, CHUNK)
    ba = b_att.reshape(1, 1)

    deg_partials = _sc_degree(col2d, dat2d).reshape(NC, N)
    deg, h1 = _tc1(deg_partials, embedding, W_item[0].T, W_att, ba)
    p1 = _sc_spmm(row1d, col1d, dat1d, deg, h1)
    f1, h2 = _tc2(p1, embedding, W_item[1].T, W_att, ba)
    p2 = _sc_spmm(row1d, col1d, dat1d, deg, h2)
    return _tc3(p2, f1)


# speedup vs baseline: 2.1158x; 1.0044x over previous
import functools

import jax
import jax.numpy as jnp
from jax import lax
from jax.experimental import pallas as pl
from jax.experimental.pallas import tpu as pltpu
from jax.experimental.pallas import tpu_sc as plsc

N = 10000
E = 320000
D = 128
L = 2

NC = 2
NS = 16
NW = NC * NS
CHUNK = 64
CPW = 160
E2 = NW * CPW * CHUNK
NB = 2
NQ = 4


def _lane_bcast(v, t):
    return lax.gather(
        v, jnp.full((16, 1), t, jnp.int32),
        dimension_numbers=lax.GatherDimensionNumbers(
            offset_dims=(), collapsed_slice_dims=(0,), start_index_map=(0,)),
        slice_sizes=(1,),
        mode=lax.GatherScatterMode.PROMISE_IN_BOUNDS)


CPD = E2 // CHUNK // NS
CPWV = E2 // CHUNK // NW


def _sc_degree_vals(col2d, dat2d):
    mesh = plsc.VectorSubcoreMesh(core_axis_name="c", subcore_axis_name="s")

    @functools.partial(
        pl.kernel,
        out_type=jax.ShapeDtypeStruct((E2 // CHUNK, CHUNK), jnp.float32),
        mesh=mesh,
        scratch_types=[
            pltpu.VMEM_SHARED((N,), jnp.float32),
            pltpu.VMEM((CPD, CHUNK), jnp.int32),
            pltpu.VMEM((CPD, CHUNK), jnp.float32),
            pltpu.VMEM((CPWV, CHUNK), jnp.float32),
            pltpu.VMEM((CHUNK,), jnp.float32),
            pltpu.VMEM((CHUNK,), jnp.float32),
            pltpu.VMEM((640,), jnp.float32),
            pltpu.SemaphoreType.DMA,
            pltpu.SemaphoreType.DMA,
            pltpu.SemaphoreType.DMA,
            pltpu.SemaphoreType.DMA,
        ],
    )
    def k(col_hbm, dat_hbm, vals_hbm, acc, colb, datb, valsb, dg0, dg1,
          zbuf, sem_p, sem_s, sem_d0, sem_d1):
        dg = (dg0, dg1)
        sem_d = (sem_d0, sem_d1)
        c = lax.axis_index("c")
        s = lax.axis_index("s")
        wid = s * NC + c
        blk = s * CPD

        p1 = pltpu.async_copy(col_hbm.at[pl.ds(blk, CPD)], colb, sem_p)
        p2 = pltpu.async_copy(dat_hbm.at[pl.ds(blk, CPD)], datb, sem_p)

        z = jnp.zeros((16,), jnp.float32)

        @pl.loop(0, 640, step=16)
        def _(i):
            zbuf[pl.ds(i, 16)] = z

        pltpu.sync_copy(zbuf.at[pl.ds(0, 624)], acc.at[pl.ds(s * 624, 624)])

        @pl.when(s == NS - 1)
        def _():
            pltpu.sync_copy(zbuf.at[pl.ds(0, 16)], acc.at[pl.ds(9984, 16)])

        p1.wait()
        p2.wait()
        plsc.subcore_barrier()

        @pl.loop(0, CPD // 8)
        def _(g):
            descs = []
            for u in range(8):
                i = g * 8 + u
                descs.append(pltpu.async_copy(
                    datb.at[i], acc.at[colb.at[i]], sem_s, add=True))
            for d in descs:
                d.wait()

        plsc.subcore_barrier()

        lo = c * CPWV

        def issue_dg(j, b):
            pltpu.async_copy(acc.at[colb.at[lo + j]], dg[b], sem_d[b])

        def wait_dg(j, b):
            pltpu.make_async_copy(acc.at[colb.at[lo + j]], dg[b],
                                  sem_d[b]).wait()

        def vals_step(j, b, more):
            wait_dg(j, b)
            for kk in range(CHUNK // 16):
                sl = pl.ds(kk * 16, 16)
                valsb[j, sl] = datb[lo + j, sl] / dg[b][sl]
            if more:
                issue_dg(j + 2, b)

        issue_dg(0, 0)
        issue_dg(1, 1)

        @pl.loop(0, CPWV // 2 - 1)
        def _(g):
            for b in range(2):
                vals_step(g * 2 + b, b, more=True)

        vals_step(CPWV - 2, 0, more=False)
        vals_step(CPWV - 1, 1, more=False)

        pltpu.sync_copy(valsb, vals_hbm.at[pl.ds(wid * CPWV, CPWV)])

    return k(col2d, dat2d)


def _sc_spmm(row1d, col1d, vals1d, h):
    mesh = plsc.VectorSubcoreMesh(core_axis_name="c", subcore_axis_name="s")

    idx_slots = [
        pltpu.VMEM((CHUNK,), jnp.int32) for _ in range(2 * NQ)
    ] + [pltpu.VMEM((CHUNK,), jnp.float32) for _ in range(NQ)]

    @functools.partial(
        pl.kernel,
        out_type=jax.ShapeDtypeStruct((NC, N, D), jnp.float32),
        mesh=mesh,
        scratch_types=[
            pltpu.VMEM_SHARED((N + 64, D), jnp.float32),
        ] + idx_slots + [
            pltpu.VMEM((CHUNK, D), jnp.float32),
            pltpu.VMEM((CHUNK, D), jnp.float32),
            pltpu.VMEM((CHUNK, D), jnp.float32),
            pltpu.VMEM((CHUNK, D), jnp.float32),
            pltpu.SemaphoreType.DMA,
            pltpu.SemaphoreType.DMA,
            pltpu.SemaphoreType.DMA,
            pltpu.SemaphoreType.DMA,
            pltpu.SemaphoreType.DMA,
            pltpu.SemaphoreType.DMA,
            pltpu.SemaphoreType.DMA,
            pltpu.SemaphoreType.DMA,
        ],
    )
    def k(row_hbm, col_hbm, vals_hbm, h_hbm, out_hbm, acc,
          rv0, rv1, rv2, rv3, cv0, cv1, cv2, cv3, dv0, dv1, dv2, dv3,
          rg0, rg1, rs0, rs1,
          sem_g0, sem_g1, sem_s0, sem_s1, si0, si1, si2, si3):
        rowv = (rv0, rv1, rv2, rv3)
        colv = (cv0, cv1, cv2, cv3)
        datv = (dv0, dv1, dv2, dv3)
        rg = (rg0, rg1)
        rs = (rs0, rs1)
        sem_g = (sem_g0, sem_g1)
        sem_s = (sem_s0, sem_s1)
        sem_i = (si0, si1, si2, si3)

        c = lax.axis_index("c")
        s = lax.axis_index("s")
        wid = s * NC + c
        base = wid * CPW * CHUNK

        def issue_idx(cc, q):
            off = base + cc * CHUNK
            pltpu.async_copy(row_hbm.at[pl.ds(off, CHUNK)], rowv[q], sem_i[q])
            pltpu.async_copy(col_hbm.at[pl.ds(off, CHUNK)], colv[q], sem_i[q])
            pltpu.async_copy(vals_hbm.at[pl.ds(off, CHUNK)], datv[q],
                             sem_i[q])

        def wait_idx(cc, q):
            off = base + cc * CHUNK
            pltpu.make_async_copy(row_hbm.at[pl.ds(off, CHUNK)], rowv[q],
                                  sem_i[q]).wait()
            pltpu.make_async_copy(col_hbm.at[pl.ds(off, CHUNK)], colv[q],
                                  sem_i[q]).wait()
            pltpu.make_async_copy(vals_hbm.at[pl.ds(off, CHUNK)], datv[q],
                                  sem_i[q]).wait()

        def issue_gather(b, q):
            pltpu.async_copy(h_hbm.at[colv[q]], rg[b], sem_g[b])

        def wait_gather(b, q):
            pltpu.make_async_copy(h_hbm.at[colv[q]], rg[b], sem_g[b]).wait()

        def issue_scatter(b, q):
            pltpu.async_copy(rs[b], acc.at[rowv[q]], sem_s[b], add=True)

        def wait_scatter(b, q):
            pltpu.make_async_copy(rs[b], acc.at[rowv[q]], sem_s[b]).wait()

        def scale_chunk(b, q):
            @pl.loop(0, CHUNK // 16)
            def _(jj):
                sj = datv[q][pl.ds(jj * 16, 16)]
                for t in range(16):
                    e = jj * 16 + t
                    vv = _lane_bcast(sj, t)
                    for d in range(D // 16):
                        rs[b][e, pl.ds(d * 16, 16)] = (
                            rg[b][e, pl.ds(d * 16, 16)] * vv)

        def chunk_step(cc, b, q, first, more):
            wait_gather(b, q)
            if not first:
                wait_scatter(b, (q + 2) % NQ)
                if more:
                    issue_idx(cc + 2, (q + 2) % NQ)
            scale_chunk(b, q)
            if more:
                wait_idx(cc + 2, (q + 2) % NQ)
                issue_gather(b, (q + 2) % NQ)
            issue_scatter(b, q)

        for q in range(NQ):
            issue_idx(q, q)
        for b in range(NB):
            wait_idx(b, b)
            issue_gather(b, b)

        z = jnp.zeros((16,), jnp.float32)

        @pl.loop(0, CHUNK)
        def _(i):
            for d in range(D // 16):
                rs0[i, pl.ds(d * 16, 16)] = z

        @pl.loop(0, 9)
        def _(i):
            pltpu.sync_copy(rs0, acc.at[pl.ds(s * 624 + i * 64, 64)])

        pltpu.sync_copy(rs0.at[pl.ds(0, 48)], acc.at[pl.ds(s * 624 + 576, 48)])

        @pl.when(s == NS - 1)
        def _():
            pltpu.sync_copy(rs0.at[pl.ds(0, 16)], acc.at[pl.ds(9984, 16)])

        plsc.subcore_barrier()

        chunk_step(0, 0, 0, first=True, more=True)
        chunk_step(1, 1, 1, first=True, more=True)

        @pl.loop(0, (CPW - 4) // NQ)
        def _(g):
            cc = 2 + g * NQ
            for kk in range(NQ):
                chunk_step(cc + kk, (2 + kk) % NB, (2 + kk) % NQ,
                           first=False, more=True)

        chunk_step(CPW - 2, 0, 2, first=False, more=False)
        chunk_step(CPW - 1, 1, 3, first=False, more=False)
        wait_scatter(0, 2)
        wait_scatter(1, 3)

        plsc.subcore_barrier()

        @pl.loop(0, 9)
        def _(i):
            r0 = s * 624 + i * 64
            pltpu.sync_copy(acc.at[pl.ds(r0, 64)], rs0)
            pltpu.sync_copy(rs0, out_hbm.at[c, pl.ds(r0, 64)])

        pltpu.sync_copy(acc.at[pl.ds(s * 624 + 576, 48)], rs0.at[pl.ds(0, 48)])
        pltpu.sync_copy(rs0.at[pl.ds(0, 48)],
                        out_hbm.at[c, pl.ds(s * 624 + 576, 48)])

        @pl.when(s == NS - 1)
        def _():
            pltpu.sync_copy(acc.at[pl.ds(9984, 16)], rs0.at[pl.ds(0, 16)])
            pltpu.sync_copy(rs0.at[pl.ds(0, 16)],
                            out_hbm.at[c, pl.ds(9984, 16)])

    return k(row1d, col1d, vals1d, h)


def _tc1_body(emb_ref, w1t_ref, wa_ref, ba_ref, h_ref):
    x = emb_ref[...]
    scores = jnp.dot(x, wa_ref[...], preferred_element_type=jnp.float32)
    scores = scores + ba_ref[0, 0]
    m = jnp.max(scores)
    ex = jnp.exp(scores - m)
    attn = ex / jnp.sum(ex)
    xw = jnp.dot(x, w1t_ref[...], preferred_element_type=jnp.float32)
    h_ref[...] = xw * attn


def _tc1(emb, w1t, wa, ba):
    return pl.pallas_call(
        _tc1_body,
        out_shape=jax.ShapeDtypeStruct((N, D), jnp.float32),
    )(emb, w1t, wa, ba)


def _norm_rows(x):
    nrm = jnp.sqrt(jnp.sum(x * x, axis=-1, keepdims=True))
    return x / jnp.maximum(nrm, 1e-12)


def _tc2_body(p_ref, emb_ref, w2t_ref, wa_ref, ba_ref, f1_ref, h_ref):
    x = _norm_rows(p_ref[0] + p_ref[1])
    f1_ref[...] = emb_ref[...] + x
    scores = jnp.dot(x, wa_ref[...], preferred_element_type=jnp.float32)
    scores = scores + ba_ref[0, 0]
    m = jnp.max(scores)
    ex = jnp.exp(scores - m)
    attn = ex / jnp.sum(ex)
    xw = jnp.dot(x, w2t_ref[...], preferred_element_type=jnp.float32)
    h_ref[...] = xw * attn


def _tc2(partials, emb, w2t, wa, ba):
    return pl.pallas_call(
        _tc2_body,
        out_shape=(jax.ShapeDtypeStruct((N, D), jnp.float32),
                   jax.ShapeDtypeStruct((N, D), jnp.float32)),
    )(partials, emb, w2t, wa, ba)


def _tc3_body(p_ref, f1_ref, out_ref):
    x = _norm_rows(p_ref[0] + p_ref[1])
    out_ref[...] = (f1_ref[...] + x) * (1.0 / (L + 1))


def _tc3(partials, f1):
    return pl.pallas_call(
        _tc3_body,
        out_shape=jax.ShapeDtypeStruct((N, D), jnp.float32),
    )(partials, f1)


def kernel(adj_row, adj_col, adj_data, embedding, W_item, W_att, b_att):
    pad = E2 - E
    trash = N + (jnp.arange(pad, dtype=jnp.int32) % 64)
    row1d = jnp.concatenate([adj_row.astype(jnp.int32), trash])
    col1d = jnp.concatenate(
        [adj_col.astype(jnp.int32), jnp.arange(pad, dtype=jnp.int32) % N])
    dat1d = jnp.concatenate([adj_data, jnp.zeros((pad,), jnp.float32)])
    col2d = col1d.reshape(E2 // CHUNK, CHUNK)
    dat2d = dat1d.reshape(E2 // CHUNK, CHUNK)
    ba = b_att.reshape(1, 1)

    vals1d = _sc_degree_vals(col2d, dat2d).reshape(E2)
    h1 = _tc1(embedding, W_item[0].T, W_att, ba)
    p1 = _sc_spmm(row1d, col1d, vals1d, h1)
    f1, h2 = _tc2(p1, embedding, W_item[1].T, W_att, ba)
    p2 = _sc_spmm(row1d, col1d, vals1d, h2)
    return _tc3(p2, f1)


# speedup vs baseline: 2.1388x; 1.0109x over previous
import functools

import jax
import jax.numpy as jnp
from jax import lax
from jax.experimental import pallas as pl
from jax.experimental.pallas import tpu as pltpu
from jax.experimental.pallas import tpu_sc as plsc

N = 10000
E = 320000
D = 128
L = 2

NC = 2
NS = 16
NW = NC * NS
CHUNK = 64
CPW = 160
E2 = NW * CPW * CHUNK
NB = 2
NQ = 4


def _lane_bcast(v, t):
    return lax.gather(
        v, jnp.full((16, 1), t, jnp.int32),
        dimension_numbers=lax.GatherDimensionNumbers(
            offset_dims=(), collapsed_slice_dims=(0,), start_index_map=(0,)),
        slice_sizes=(1,),
        mode=lax.GatherScatterMode.PROMISE_IN_BOUNDS)


CPD = E2 // CHUNK // NS
CPWV = E2 // CHUNK // NW


def _sc_degree_vals(col2d, dat2d):
    mesh = plsc.VectorSubcoreMesh(core_axis_name="c", subcore_axis_name="s")

    @functools.partial(
        pl.kernel,
        out_type=jax.ShapeDtypeStruct((E2 // CHUNK, CHUNK), jnp.float32),
        mesh=mesh,
        scratch_types=[
            pltpu.VMEM_SHARED((N,), jnp.float32),
            pltpu.VMEM((CPD, CHUNK), jnp.int32),
            pltpu.VMEM((CPD, CHUNK), jnp.float32),
            pltpu.VMEM((CPWV, CHUNK), jnp.float32),
            pltpu.VMEM((CHUNK,), jnp.float32),
            pltpu.VMEM((CHUNK,), jnp.float32),
            pltpu.VMEM((640,), jnp.float32),
            pltpu.SemaphoreType.DMA,
            pltpu.SemaphoreType.DMA,
            pltpu.SemaphoreType.DMA,
            pltpu.SemaphoreType.DMA,
        ],
    )
    def k(col_hbm, dat_hbm, vals_hbm, acc, colb, datb, valsb, dg0, dg1,
          zbuf, sem_p, sem_s, sem_d0, sem_d1):
        dg = (dg0, dg1)
        sem_d = (sem_d0, sem_d1)
        c = lax.axis_index("c")
        s = lax.axis_index("s")
        wid = s * NC + c
        blk = s * CPD

        p1 = pltpu.async_copy(col_hbm.at[pl.ds(blk, CPD)], colb, sem_p)
        p2 = pltpu.async_copy(dat_hbm.at[pl.ds(blk, CPD)], datb, sem_p)

        z = jnp.zeros((16,), jnp.float32)

        @pl.loop(0, 640, step=16)
        def _(i):
            zbuf[pl.ds(i, 16)] = z

        pltpu.sync_copy(zbuf.at[pl.ds(0, 624)], acc.at[pl.ds(s * 624, 624)])

        @pl.when(s == NS - 1)
        def _():
            pltpu.sync_copy(zbuf.at[pl.ds(0, 16)], acc.at[pl.ds(9984, 16)])

        p1.wait()
        p2.wait()
        plsc.subcore_barrier()

        @pl.loop(0, CPD // 8)
        def _(g):
            descs = []
            for u in range(8):
                i = g * 8 + u
                descs.append(pltpu.async_copy(
                    datb.at[i], acc.at[colb.at[i]], sem_s, add=True))
            for d in descs:
                d.wait()

        plsc.subcore_barrier()

        lo = c * CPWV

        def issue_dg(j, b):
            pltpu.async_copy(acc.at[colb.at[lo + j]], dg[b], sem_d[b])

        def wait_dg(j, b):
            pltpu.make_async_copy(acc.at[colb.at[lo + j]], dg[b],
                                  sem_d[b]).wait()

        def vals_step(j, b, more):
            wait_dg(j, b)
            for kk in range(CHUNK // 16):
                sl = pl.ds(kk * 16, 16)
                valsb[j, sl] = datb[lo + j, sl] / dg[b][sl]
            if more:
                issue_dg(j + 2, b)

        issue_dg(0, 0)
        issue_dg(1, 1)

        @pl.loop(0, CPWV // 2 - 1)
        def _(g):
            for b in range(2):
                vals_step(g * 2 + b, b, more=True)

        vals_step(CPWV - 2, 0, more=False)
        vals_step(CPWV - 1, 1, more=False)

        pltpu.sync_copy(valsb, vals_hbm.at[pl.ds(wid * CPWV, CPWV)])

    return k(col2d, dat2d)


def _sc_spmm(row1d, col1d, vals1d, h):
    mesh = plsc.VectorSubcoreMesh(core_axis_name="c", subcore_axis_name="s")

    idx_slots = [
        pltpu.VMEM((CHUNK,), jnp.int32) for _ in range(2 * NQ)
    ] + [pltpu.VMEM((CHUNK,), jnp.float32) for _ in range(NQ)]

    @functools.partial(
        pl.kernel,
        out_type=jax.ShapeDtypeStruct((NC, N, D), jnp.float32),
        mesh=mesh,
        scratch_types=[
            pltpu.VMEM_SHARED((N + 64, D), jnp.float32),
        ] + idx_slots + [
            pltpu.VMEM((CHUNK, D), jnp.float32),
            pltpu.VMEM((CHUNK, D), jnp.float32),
            pltpu.VMEM((CHUNK, D), jnp.float32),
            pltpu.VMEM((CHUNK, D), jnp.float32),
            pltpu.SemaphoreType.DMA,
            pltpu.SemaphoreType.DMA,
            pltpu.SemaphoreType.DMA,
            pltpu.SemaphoreType.DMA,
            pltpu.SemaphoreType.DMA,
            pltpu.SemaphoreType.DMA,
            pltpu.SemaphoreType.DMA,
            pltpu.SemaphoreType.DMA,
        ],
    )
    def k(row_hbm, col_hbm, vals_hbm, h_hbm, out_hbm, acc,
          rv0, rv1, rv2, rv3, cv0, cv1, cv2, cv3, dv0, dv1, dv2, dv3,
          rg0, rg1, rs0, rs1,
          sem_g0, sem_g1, sem_s0, sem_s1, si0, si1, si2, si3):
        rowv = (rv0, rv1, rv2, rv3)
        colv = (cv0, cv1, cv2, cv3)
        datv = (dv0, dv1, dv2, dv3)
        rg = (rg0, rg1)
        rs = (rs0, rs1)
        sem_g = (sem_g0, sem_g1)
        sem_s = (sem_s0, sem_s1)
        sem_i = (si0, si1, si2, si3)

        c = lax.axis_index("c")
        s = lax.axis_index("s")
        wid = s * NC + c
        base = wid * CPW * CHUNK

        def issue_idx(cc, q):
            off = base + cc * CHUNK
            pltpu.async_copy(row_hbm.at[pl.ds(off, CHUNK)], rowv[q], sem_i[q])
            pltpu.async_copy(col_hbm.at[pl.ds(off, CHUNK)], colv[q], sem_i[q])
            pltpu.async_copy(vals_hbm.at[pl.ds(off, CHUNK)], datv[q],
                             sem_i[q])

        def wait_idx(cc, q):
            off = base + cc * CHUNK
            pltpu.make_async_copy(row_hbm.at[pl.ds(off, CHUNK)], rowv[q],
                                  sem_i[q]).wait()
            pltpu.make_async_copy(col_hbm.at[pl.ds(off, CHUNK)], colv[q],
                                  sem_i[q]).wait()
            pltpu.make_async_copy(vals_hbm.at[pl.ds(off, CHUNK)], datv[q],
                                  sem_i[q]).wait()

        def issue_gather(b, q):
            pltpu.async_copy(h_hbm.at[colv[q]], rg[b], sem_g[b])

        def wait_gather(b, q):
            pltpu.make_async_copy(h_hbm.at[colv[q]], rg[b], sem_g[b]).wait()

        def issue_scatter(b, q):
            pltpu.async_copy(rs[b], acc.at[rowv[q]], sem_s[b], add=True)

        def wait_scatter(b, q):
            pltpu.make_async_copy(rs[b], acc.at[rowv[q]], sem_s[b]).wait()

        def scale_chunk(b, q):
            @pl.loop(0, CHUNK // 16)
            def _(jj):
                sj = datv[q][pl.ds(jj * 16, 16)]
                for t in range(16):
                    e = jj * 16 + t
                    vv = _lane_bcast(sj, t)
                    for d in range(D // 16):
                        rs[b][e, pl.ds(d * 16, 16)] = (
                            rg[b][e, pl.ds(d * 16, 16)] * vv)

        def chunk_step(cc, b, q, first, more):
            wait_gather(b, q)
            if not first:
                wait_scatter(b, (q + 2) % NQ)
                if more:
                    issue_idx(cc + 2, (q + 2) % NQ)
            scale_chunk(b, q)
            if more:
                wait_idx(cc + 2, (q + 2) % NQ)
                issue_gather(b, (q + 2) % NQ)
            issue_scatter(b, q)

        for q in range(NQ):
            issue_idx(q, q)
        for b in range(NB):
            wait_idx(b, b)
            issue_gather(b, b)

        z = jnp.zeros((16,), jnp.float32)

        @pl.loop(0, CHUNK)
        def _(i):
            for d in range(D // 16):
                rs0[i, pl.ds(d * 16, 16)] = z

        @pl.loop(0, 9)
        def _(i):
            pltpu.sync_copy(rs0, acc.at[pl.ds(s * 624 + i * 64, 64)])

        pltpu.sync_copy(rs0.at[pl.ds(0, 48)], acc.at[pl.ds(s * 624 + 576, 48)])

        @pl.when(s == NS - 1)
        def _():
            pltpu.sync_copy(rs0.at[pl.ds(0, 16)], acc.at[pl.ds(9984, 16)])

        plsc.subcore_barrier()

        chunk_step(0, 0, 0, first=True, more=True)
        chunk_step(1, 1, 1, first=True, more=True)

        @pl.loop(0, (CPW - 4) // NQ)
        def _(g):
            cc = 2 + g * NQ
            for kk in range(NQ):
                chunk_step(cc + kk, (2 + kk) % NB, (2 + kk) % NQ,
                           first=False, more=True)

        chunk_step(CPW - 2, 0, 2, first=False, more=False)
        chunk_step(CPW - 1, 1, 3, first=False, more=False)
        wait_scatter(0, 2)
        wait_scatter(1, 3)

        plsc.subcore_barrier()

        def cp_in(i, buf, sem):
            return pltpu.async_copy(acc.at[pl.ds(s * 624 + i * 64, 64)],
                                    buf, sem)

        def cp_out(i, buf, sem):
            return pltpu.async_copy(buf, out_hbm.at[c, pl.ds(s * 624 + i * 64,
                                                             64)], sem)

        def w_in(i, buf, sem):
            pltpu.make_async_copy(acc.at[pl.ds(s * 624 + i * 64, 64)],
                                  buf, sem).wait()

        def w_out(i, buf, sem):
            pltpu.make_async_copy(buf, out_hbm.at[c, pl.ds(s * 624 + i * 64,
                                                           64)], sem).wait()

        cbuf = (rs0, rs1)
        csem = (sem_g0, sem_g1)
        osem = (sem_s0, sem_s1)
        for i in range(9):
            b = i % 2
            if i >= 2:
                w_out(i - 2, cbuf[b], osem[b])
            cp_in(i, cbuf[b], csem[b])
            w_in(i, cbuf[b], csem[b])
            cp_out(i, cbuf[b], osem[b])
        w_out(7, cbuf[1], osem[1])
        w_out(8, cbuf[0], osem[0])

        pltpu.sync_copy(acc.at[pl.ds(s * 624 + 576, 48)], rs0.at[pl.ds(0, 48)])
        pltpu.sync_copy(rs0.at[pl.ds(0, 48)],
                        out_hbm.at[c, pl.ds(s * 624 + 576, 48)])

        @pl.when(s == NS - 1)
        def _():
            pltpu.sync_copy(acc.at[pl.ds(9984, 16)], rs0.at[pl.ds(0, 16)])
            pltpu.sync_copy(rs0.at[pl.ds(0, 16)],
                            out_hbm.at[c, pl.ds(9984, 16)])

    return k(row1d, col1d, vals1d, h)


def _tc1_body(emb_ref, w1t_ref, wa_ref, ba_ref, h_ref):
    x = emb_ref[...]
    scores = jnp.dot(x, wa_ref[...], preferred_element_type=jnp.float32)
    scores = scores + ba_ref[0, 0]
    m = jnp.max(scores)
    ex = jnp.exp(scores - m)
    attn = ex / jnp.sum(ex)
    xw = jnp.dot(x, w1t_ref[...], preferred_element_type=jnp.float32)
    h_ref[...] = xw * attn


def _tc1(emb, w1t, wa, ba):
    return pl.pallas_call(
        _tc1_body,
        out_shape=jax.ShapeDtypeStruct((N, D), jnp.float32),
    )(emb, w1t, wa, ba)


def _norm_rows(x):
    nrm = jnp.sqrt(jnp.sum(x * x, axis=-1, keepdims=True))
    return x / jnp.maximum(nrm, 1e-12)


def _tc2_body(p_ref, emb_ref, w2t_ref, wa_ref, ba_ref, f1_ref, h_ref):
    x = _norm_rows(p_ref[0] + p_ref[1])
    f1_ref[...] = emb_ref[...] + x
    scores = jnp.dot(x, wa_ref[...], preferred_element_type=jnp.float32)
    scores = scores + ba_ref[0, 0]
    m = jnp.max(scores)
    ex = jnp.exp(scores - m)
    attn = ex / jnp.sum(ex)
    xw = jnp.dot(x, w2t_ref[...], preferred_element_type=jnp.float32)
    h_ref[...] = xw * attn


def _tc2(partials, emb, w2t, wa, ba):
    return pl.pallas_call(
        _tc2_body,
        out_shape=(jax.ShapeDtypeStruct((N, D), jnp.float32),
                   jax.ShapeDtypeStruct((N, D), jnp.float32)),
    )(partials, emb, w2t, wa, ba)


def _tc3_body(p_ref, f1_ref, out_ref):
    x = _norm_rows(p_ref[0] + p_ref[1])
    out_ref[...] = (f1_ref[...] + x) * (1.0 / (L + 1))


def _tc3(partials, f1):
    return pl.pallas_call(
        _tc3_body,
        out_shape=jax.ShapeDtypeStruct((N, D), jnp.float32),
    )(partials, f1)


def kernel(adj_row, adj_col, adj_data, embedding, W_item, W_att, b_att):
    pad = E2 - E
    trash = N + (jnp.arange(pad, dtype=jnp.int32) % 64)
    row1d = jnp.concatenate([adj_row.astype(jnp.int32), trash])
    col1d = jnp.concatenate(
        [adj_col.astype(jnp.int32), jnp.arange(pad, dtype=jnp.int32) % N])
    dat1d = jnp.concatenate([adj_data, jnp.zeros((pad,), jnp.float32)])
    col2d = col1d.reshape(E2 // CHUNK, CHUNK)
    dat2d = dat1d.reshape(E2 // CHUNK, CHUNK)
    ba = b_att.reshape(1, 1)

    vals1d = _sc_degree_vals(col2d, dat2d).reshape(E2)
    h1 = _tc1(embedding, W_item[0].T, W_att, ba)
    p1 = _sc_spmm(row1d, col1d, vals1d, h1)
    f1, h2 = _tc2(p1, embedding, W_item[1].T, W_att, ba)
    p2 = _sc_spmm(row1d, col1d, vals1d, h2)
    return _tc3(p2, f1)
